# baseline (speedup 1.0000x reference)
import functools

import jax
import jax.numpy as jnp
from jax import lax
from jax.experimental import pallas as pl
from jax.experimental.pallas import tpu as pltpu
from jax.experimental.pallas import tpu_sc as plsc

B = 1024
D = 8192
NC = 2
NS = 16
L = 16
NW = NC * NS
BPW = B // NW
CHUNK = 2
NROUND = BPW // CHUNK
VPR = D // L

_mesh = plsc.VectorSubcoreMesh(
    core_axis_name="c", subcore_axis_name="s", num_cores=NC, num_subcores=NS)


@functools.partial(
    pl.kernel,
    out_type=jax.ShapeDtypeStruct((NW, L), jnp.float32),
    mesh=_mesh,
    scratch_types=[
        pltpu.VMEM((NROUND, CHUNK), jnp.int32),
        pltpu.VMEM((CHUNK, D), jnp.float32),
        pltpu.VMEM((CHUNK, D), jnp.float32),
        pltpu.VMEM((L,), jnp.float32),
        pltpu.SemaphoreType.DMA,
        pltpu.SemaphoreType.DMA,
    ],
)
def _center_loss_partials(feat_hbm, lab_hbm, cent_hbm, out_hbm,
                          idx_v, fbuf, cbuf, accv, fsem, csem):
    wid = lax.axis_index("s") * NC + lax.axis_index("c")
    base = wid * BPW
    pltpu.sync_copy(lab_hbm.at[wid], idx_v)

    def round_body(g, acc):
        fcp = pltpu.make_async_copy(
            feat_hbm.at[pl.ds(base + g * CHUNK, CHUNK)], fbuf, fsem)
        fcp.start()
        ccp = pltpu.make_async_copy(cent_hbm.at[idx_v.at[g]], cbuf, csem)
        ccp.start()
        fcp.wait()
        ccp.wait()
        for r in range(CHUNK):
            def vec_body(k, a, r=r):
                f = fbuf[r, pl.ds(k * L, L)]
                c = cbuf[r, pl.ds(k * L, L)]
                dd = f - c
                return a + dd * dd
            acc = lax.fori_loop(0, VPR, vec_body, acc)
        return acc

    acc = lax.fori_loop(0, NROUND, round_body, jnp.zeros((L,), jnp.float32))
    accv[...] = acc
    pltpu.sync_copy(accv, out_hbm.at[wid])


def kernel(features, labels, centers):
    lab = labels.astype(jnp.int32).reshape(NW, NROUND, CHUNK)
    partials = _center_loss_partials(features, lab, centers)
    return 0.5 * jnp.sum(partials) / features.shape[0]


# speedup vs baseline: 2.1553x; 2.1553x over previous
import functools

import jax
import jax.numpy as jnp
from jax import lax
from jax.experimental import pallas as pl
from jax.experimental.pallas import tpu as pltpu
from jax.experimental.pallas import tpu_sc as plsc

B = 1024
D = 8192
NC = 2
NS = 16
L = 16
NW = NC * NS
BPW = B // NW
CHUNK = 2
NROUND = BPW // CHUNK
NBUF = 2
NVEC = 8

_mesh = plsc.VectorSubcoreMesh(
    core_axis_name="c", subcore_axis_name="s", num_cores=NC, num_subcores=NS)


@functools.partial(
    pl.kernel,
    out_type=jax.ShapeDtypeStruct((NW, L), jnp.float32),
    mesh=_mesh,
    scratch_types=[
        pltpu.VMEM((NROUND, CHUNK), jnp.int32),
        pltpu.VMEM((NBUF, CHUNK, D), jnp.float32),
        pltpu.VMEM((NBUF, CHUNK, D), jnp.float32),
        pltpu.VMEM((L,), jnp.float32),
        pltpu.SemaphoreType.DMA((NBUF,)),
        pltpu.SemaphoreType.DMA((NBUF,)),
    ],
)
def _center_loss_partials(feat_hbm, lab_hbm, cent_hbm, out_hbm,
                          idx_v, fbuf, cbuf, accv, fsems, csems):
    wid = lax.axis_index("s") * NC + lax.axis_index("c")
    base = wid * BPW
    pltpu.sync_copy(lab_hbm.at[wid], idx_v)

    def start(g, b):
        pltpu.make_async_copy(
            feat_hbm.at[pl.ds(base + g * CHUNK, CHUNK)], fbuf.at[b],
            fsems.at[b]).start()
        pltpu.make_async_copy(
            cent_hbm.at[idx_v.at[g]], cbuf.at[b], csems.at[b]).start()

    def wait(b):
        pltpu.make_async_copy(
            feat_hbm.at[pl.ds(0, CHUNK)], fbuf.at[b], fsems.at[b]).wait()
        pltpu.make_async_copy(
            cent_hbm.at[idx_v.at[0]], cbuf.at[b], csems.at[b]).wait()

    def compute(b, r, accs):
        def vbody(i, accs):
            f = [fbuf[b, r, pl.ds(i + j * L, L)] for j in range(NVEC)]
            c = [cbuf[b, r, pl.ds(i + j * L, L)] for j in range(NVEC)]
            d = [f[j] - c[j] for j in range(NVEC)]
            return tuple(accs[j] + d[j] * d[j] for j in range(NVEC))
        return plsc.parallel_loop(0, D, step=NVEC * L, carry=accs)(vbody)

    for b in range(NBUF):
        start(b, b)

    def outer(t, accs):
        for b in range(NBUF):
            g = t * NBUF + b
            wait(b)
            for r in range(CHUNK):
                accs = compute(b, r, accs)

            @pl.when(g + NBUF < NROUND)
            def _():
                start(g + NBUF, b)
        return accs

    zero = jnp.zeros((L,), jnp.float32)
    accs = lax.fori_loop(0, NROUND // NBUF, outer, (zero,) * NVEC)
    acc = accs[0]
    for j in range(1, NVEC):
        acc = acc + accs[j]
    accv[...] = acc
    pltpu.sync_copy(accv, out_hbm.at[wid])


def kernel(features, labels, centers):
    lab = labels.astype(jnp.int32).reshape(NW, NROUND, CHUNK)
    partials = _center_loss_partials(features, lab, centers)
    return 0.5 * jnp.sum(partials) / features.shape[0]


# speedup vs baseline: 2.3316x; 1.0818x over previous
import functools

import jax
import jax.numpy as jnp
from jax import lax
from jax.experimental import pallas as pl
from jax.experimental.pallas import tpu as pltpu
from jax.experimental.pallas import tpu_sc as plsc

B = 1024
D = 8192
NC = 2
NS = 16
L = 16
NW = NC * NS
BPW = B // NW
CHUNK = 1
NROUND = BPW // CHUNK
NBUF = 4
NVEC = 8

_mesh = plsc.VectorSubcoreMesh(
    core_axis_name="c", subcore_axis_name="s", num_cores=NC, num_subcores=NS)


@functools.partial(
    pl.kernel,
    out_type=jax.ShapeDtypeStruct((NW, L), jnp.float32),
    mesh=_mesh,
    scratch_types=[
        pltpu.VMEM((NROUND, CHUNK), jnp.int32),
        pltpu.VMEM((NBUF, CHUNK, D), jnp.float32),
        pltpu.VMEM((NBUF, CHUNK, D), jnp.float32),
        pltpu.VMEM((L,), jnp.float32),
        pltpu.SemaphoreType.DMA((NBUF,)),
        pltpu.SemaphoreType.DMA((NBUF,)),
    ],
)
def _center_loss_partials(feat_hbm, lab_hbm, cent_hbm, out_hbm,
                          idx_v, fbuf, cbuf, accv, fsems, csems):
    wid = lax.axis_index("s") * NC + lax.axis_index("c")
    base = wid * BPW
    pltpu.sync_copy(lab_hbm.at[wid], idx_v)

    def start(g, b):
        pltpu.make_async_copy(
            feat_hbm.at[pl.ds(base + g * CHUNK, CHUNK)], fbuf.at[b],
            fsems.at[b]).start()
        pltpu.make_async_copy(
            cent_hbm.at[idx_v.at[g]], cbuf.at[b], csems.at[b]).start()

    def wait(b):
        pltpu.make_async_copy(
            feat_hbm.at[pl.ds(0, CHUNK)], fbuf.at[b], fsems.at[b]).wait()
        pltpu.make_async_copy(
            cent_hbm.at[idx_v.at[0]], cbuf.at[b], csems.at[b]).wait()

    def compute(b, r, accs):
        def vbody(i, accs):
            f = [fbuf[b, r, pl.ds(i + j * L, L)] for j in range(NVEC)]
            c = [cbuf[b, r, pl.ds(i + j * L, L)] for j in range(NVEC)]
            d = [f[j] - c[j] for j in range(NVEC)]
            return tuple(accs[j] + d[j] * d[j] for j in range(NVEC))
        return plsc.parallel_loop(0, D, step=NVEC * L, carry=accs)(vbody)

    for b in range(NBUF):
        start(b, b)

    def outer(t, accs):
        for b in range(NBUF):
            g = t * NBUF + b
            wait(b)
            for r in range(CHUNK):
                accs = compute(b, r, accs)

            @pl.when(g + NBUF < NROUND)
            def _():
                start(g + NBUF, b)
        return accs

    zero = jnp.zeros((L,), jnp.float32)
    accs = lax.fori_loop(0, NROUND // NBUF, outer, (zero,) * NVEC)
    acc = accs[0]
    for j in range(1, NVEC):
        acc = acc + accs[j]
    accv[...] = acc
    pltpu.sync_copy(accv, out_hbm.at[wid])


def kernel(features, labels, centers):
    lab = labels.astype(jnp.int32).reshape(NW, NROUND, CHUNK)
    partials = _center_loss_partials(features, lab, centers)
    return 0.5 * jnp.sum(partials) / features.shape[0]
